# baseline (speedup 1.0000x reference)
import functools

import jax
import jax.numpy as jnp
from jax.experimental import pallas as pl

_NK = 20
_EPS = 1e-06


def _knn_body(x_ref, xt_ref, idx_ref, *, n):
    xr0 = x_ref[0, 0:1, :]
    xr1 = x_ref[0, 1:2, :]
    xr2 = x_ref[0, 2:3, :]
    xc0 = xt_ref[0, :, 0:1]
    xc1 = xt_ref[0, :, 1:2]
    xc2 = xt_ref[0, :, 2:3]
    sq_row = xr0 * xr0 + xr1 * xr1 + xr2 * xr2
    sq_col = xc0 * xc0 + xc1 * xc1 + xc2 * xc2
    xb16 = x_ref[0].astype(jnp.bfloat16)
    cross = jax.lax.dot_general(xb16, xb16, (((0,), (0,)), ((), ())),
                                preferred_element_type=jnp.float32)
    dist = (sq_col + sq_row) - 2.0 * cross
    row_i = jax.lax.broadcasted_iota(jnp.int32, (n, n), 0)
    col_i = jax.lax.broadcasted_iota(jnp.int32, (n, n), 1)
    neg = -dist + jnp.where(row_i == col_i, 1e3, 0.0).astype(jnp.float32)
    cols = []
    for _ in range(_NK):
        amax = jnp.argmax(neg, axis=1, keepdims=True).astype(jnp.int32)
        cols.append(amax)
        neg = jnp.where(col_i == amax, -1e30, neg)
    cols.append(jnp.zeros((n, 128 - _NK), jnp.int32))
    idx_ref[0] = jnp.concatenate(cols, axis=1)


def _conv_body(idx_ref, f_ref, x_ref, x8_ref, w7_ref, kp_ref, wr_ref, wo_ref,
               bmlp_ref, bconv_ref, bout_ref, o_ref, *, n, c, out_c, t_sz):
    t = pl.program_id(1)
    idx_tile = idx_ref[0]
    fb16 = f_ref[0].astype(jnp.bfloat16)
    x_rep = x_ref[0, pl.ds(t * t_sz, t_sz), :]
    ft_tile = f_ref[0, pl.ds(t * t_sz, t_sz), :]
    w7_16 = w7_ref[...].astype(jnp.bfloat16)
    kp_16 = kp_ref[...].astype(jnp.bfloat16)
    xrow0 = x8_ref[0, 0:1, :]
    xrow1 = x8_ref[0, 1:2, :]
    xrow2 = x8_ref[0, 2:3, :]
    lane_n = jax.lax.broadcasted_iota(jnp.int32, (t_sz, n), 1)
    lane_8 = jax.lax.broadcasted_iota(jnp.int32, (t_sz, 8), 1)
    e0 = jnp.where(lane_8 == 0, 1.0, 0.0).astype(jnp.float32)
    zcol5 = jnp.zeros((t_sz, 4), jnp.float32)

    xrels = []
    ohs = []
    pm1 = []
    s1 = jnp.zeros((t_sz, 8), jnp.float32)
    for k in range(_NK):
        oh = jnp.where(lane_n == idx_tile[:, k:k + 1], 1.0, 0.0)
        ohs.append(oh.astype(jnp.bfloat16))
        g0 = jnp.sum(oh * xrow0, axis=1, keepdims=True)
        g1 = jnp.sum(oh * xrow1, axis=1, keepdims=True)
        g2 = jnp.sum(oh * xrow2, axis=1, keepdims=True)
        gx = jnp.concatenate([g0, g1, g2, jnp.zeros((t_sz, 5), jnp.float32)],
                             axis=1)
        xr = gx - x_rep
        xrels.append(xr)
        pm = jax.lax.dot_general(xr.astype(jnp.bfloat16), kp_16,
                                 (((1,), (0,)), ((), ())),
                                 preferred_element_type=jnp.float32)
        if k == 0:
            pm = pm + e0
        pm = jnp.maximum(pm, 0.0)
        pm1.append(pm)
        s1 = s1 + pm
    den1 = s1 + _EPS
    pm2 = []
    s2 = jnp.zeros((t_sz, 8), jnp.float32)
    for k in range(_NK):
        p = pm1[k] / den1
        p = p * p
        pm2.append(p)
        s2 = s2 + p
    den2 = s2 + _EPS

    ya = [jnp.zeros((t_sz, c), jnp.float32) for _ in range(8)]
    yb = [jnp.zeros((t_sz, c), jnp.float32) for _ in range(8)]
    bmlp = bmlp_ref[0:1, :]
    dnum = (((1,), (0,)), ((), ()))
    for k in range(_NK):
        p = pm2[k] / den2
        p = jnp.where(p > 0.1, p, 0.0)
        xr = xrels[k]
        dis = jnp.sqrt(jnp.maximum(
            jnp.sum(xr * xr, axis=1, keepdims=True), 1e-12))
        feat7 = jnp.concatenate([x_rep[:, 0:3], xr[:, 0:3], dis, zcol5[:, 0:1]],
                                axis=1)
        xf = jax.lax.dot_general(feat7.astype(jnp.bfloat16), w7_16, dnum,
                                 preferred_element_type=jnp.float32) + bmlp
        gf = jnp.dot(ohs[k], fb16, preferred_element_type=jnp.float32)
        xf = xf.astype(jnp.bfloat16).astype(jnp.float32)
        p = p.astype(jnp.bfloat16).astype(jnp.float32)
        for j in range(8):
            pj = p[:, j:j + 1]
            ya[j] = ya[j] + gf * pj
            yb[j] = yb[j] + xf * pj

    acc = jnp.zeros((t_sz, out_c), jnp.float32)
    for j in range(8):
        acc = acc + jax.lax.dot_general(
            ya[j].astype(jnp.bfloat16),
            wr_ref[j * 2 * c:j * 2 * c + c, :].astype(jnp.bfloat16),
            dnum, preferred_element_type=jnp.float32)
        acc = acc + jax.lax.dot_general(
            yb[j].astype(jnp.bfloat16),
            wr_ref[j * 2 * c + c:(j + 1) * 2 * c, :].astype(jnp.bfloat16),
            dnum, preferred_element_type=jnp.float32)
    acc = acc + bconv_ref[0:1, :]
    acc = acc + jax.lax.dot_general(ft_tile.astype(jnp.bfloat16),
                                    wo_ref[...].astype(jnp.bfloat16),
                                    dnum, preferred_element_type=jnp.float32)
    acc = acc + bout_ref[0:1, :]
    o_ref[0] = acc


def _bn_body(x_ref, g_ref, b_ref, o_ref, *, cnt):
    xv = x_ref[...]
    s = jnp.sum(jnp.sum(xv, axis=1, keepdims=True), axis=0, keepdims=True)
    m = s / cnt
    d = xv - m
    v = jnp.sum(jnp.sum(d * d, axis=1, keepdims=True), axis=0,
                keepdims=True) / cnt
    g = jnp.reshape(g_ref[0:1, :], (1, 1, -1))
    bb = jnp.reshape(b_ref[0:1, :], (1, 1, -1))
    o_ref[...] = d / jnp.sqrt(v + 1e-5) * g + bb


def kernel(x, feature, W_mlp, b_mlp, W_conv, b_conv, W_out, b_out,
           gamma, beta, kernals):
    bsz, _, n = x.shape
    c = feature.shape[1]
    out_c = W_out.shape[0]
    ks = kernals.shape[1]
    t_sz = 128
    nt = n // t_sz

    x_p8 = jnp.pad(x, ((0, 0), (0, 5), (0, 0)))
    x_t = jnp.transpose(x_p8, (0, 2, 1))
    f_t = jnp.transpose(feature, (0, 2, 1))
    w7 = jnp.pad(W_mlp, ((0, 0), (0, 1))).T
    kp = jnp.pad(kernals, ((0, 5), (0, 0)))
    wr = jnp.transpose(W_conv.reshape(out_c, 2 * c, ks),
                       (2, 1, 0)).reshape(ks * 2 * c, out_c)
    wo = W_out.T
    row8 = lambda v: jnp.broadcast_to(v[None, :], (8, v.shape[0]))

    idx = pl.pallas_call(
        functools.partial(_knn_body, n=n),
        grid=(bsz,),
        in_specs=[pl.BlockSpec((1, 8, n), lambda b: (b, 0, 0)),
                  pl.BlockSpec((1, n, 8), lambda b: (b, 0, 0))],
        out_specs=pl.BlockSpec((1, n, 128), lambda b: (b, 0, 0)),
        out_shape=jax.ShapeDtypeStruct((bsz, n, 128), jnp.int32),
    )(x_p8, x_t)

    out_pre = pl.pallas_call(
        functools.partial(_conv_body, n=n, c=c, out_c=out_c, t_sz=t_sz),
        grid=(bsz, nt),
        in_specs=[
            pl.BlockSpec((1, t_sz, 128), lambda b, t: (b, t, 0)),
            pl.BlockSpec((1, n, c), lambda b, t: (b, 0, 0)),
            pl.BlockSpec((1, n, 8), lambda b, t: (b, 0, 0)),
            pl.BlockSpec((1, 8, n), lambda b, t: (b, 0, 0)),
            pl.BlockSpec((8, c), lambda b, t: (0, 0)),
            pl.BlockSpec((8, ks), lambda b, t: (0, 0)),
            pl.BlockSpec((ks * 2 * c, out_c), lambda b, t: (0, 0)),
            pl.BlockSpec((c, out_c), lambda b, t: (0, 0)),
            pl.BlockSpec((8, c), lambda b, t: (0, 0)),
            pl.BlockSpec((8, out_c), lambda b, t: (0, 0)),
            pl.BlockSpec((8, out_c), lambda b, t: (0, 0)),
        ],
        out_specs=pl.BlockSpec((1, t_sz, out_c), lambda b, t: (b, t, 0)),
        out_shape=jax.ShapeDtypeStruct((bsz, n, out_c), jnp.float32),
    )(idx, f_t, x_t, x_p8, w7, kp, wr, wo,
      row8(b_mlp), row8(b_conv), row8(b_out))

    out_n = pl.pallas_call(
        functools.partial(_bn_body, cnt=float(bsz * n)),
        in_specs=[
            pl.BlockSpec((bsz, n, out_c), lambda: (0, 0, 0)),
            pl.BlockSpec((8, out_c), lambda: (0, 0)),
            pl.BlockSpec((8, out_c), lambda: (0, 0)),
        ],
        out_specs=pl.BlockSpec((bsz, n, out_c), lambda: (0, 0, 0)),
        out_shape=jax.ShapeDtypeStruct((bsz, n, out_c), jnp.float32),
    )(out_pre, row8(gamma), row8(beta))

    return jnp.transpose(out_n, (0, 2, 1))


# speedup vs baseline: 1.0647x; 1.0647x over previous
import functools

import jax
import jax.numpy as jnp
from jax.experimental import pallas as pl

_NK = 20
_EPS = 1e-06


def _knn_body(x_ref, xt_ref, idx_ref, *, n):
    xr0 = x_ref[0, 0:1, :]
    xr1 = x_ref[0, 1:2, :]
    xr2 = x_ref[0, 2:3, :]
    xc0 = xt_ref[0, :, 0:1]
    xc1 = xt_ref[0, :, 1:2]
    xc2 = xt_ref[0, :, 2:3]
    sq_row = xr0 * xr0 + xr1 * xr1 + xr2 * xr2
    sq_col = xc0 * xc0 + xc1 * xc1 + xc2 * xc2
    xb16 = x_ref[0].astype(jnp.bfloat16)
    cross = jax.lax.dot_general(xb16, xb16, (((0,), (0,)), ((), ())),
                                preferred_element_type=jnp.float32)
    dist = (sq_col + sq_row) - 2.0 * cross
    row_i = jax.lax.broadcasted_iota(jnp.int32, (n, n), 0)
    col_i = jax.lax.broadcasted_iota(jnp.int32, (n, n), 1)
    neg = -dist + jnp.where(row_i == col_i, 1e3, 0.0).astype(jnp.float32)
    cols = []
    for _ in range(_NK):
        amax = jnp.argmax(neg, axis=1, keepdims=True).astype(jnp.int32)
        cols.append(amax)
        neg = jnp.where(col_i == amax, -1e30, neg)
    cols.append(jnp.zeros((n, 128 - _NK), jnp.int32))
    idx_ref[0] = jnp.concatenate(cols, axis=1)


def _conv_body(idx_ref, f_ref, x_ref, x8_ref, w7_ref, kp_ref, wr_ref, wo_ref,
               bmlp_ref, bconv_ref, bout_ref, o_ref, *, n, c, out_c, t_sz):
    t = pl.program_id(1)
    idx_tile = idx_ref[0]
    fb16 = f_ref[0].astype(jnp.bfloat16)
    x_rep = x_ref[0, pl.ds(t * t_sz, t_sz), :]
    ft_tile = f_ref[0, pl.ds(t * t_sz, t_sz), :]
    w7_16 = w7_ref[...].astype(jnp.bfloat16)
    kp_16 = kp_ref[...].astype(jnp.bfloat16)
    xrow0 = x8_ref[0, 0:1, :]
    xrow1 = x8_ref[0, 1:2, :]
    xrow2 = x8_ref[0, 2:3, :]
    lane_n = jax.lax.broadcasted_iota(jnp.int32, (t_sz, n), 1)
    lane_8 = jax.lax.broadcasted_iota(jnp.int32, (t_sz, 8), 1)
    e0 = jnp.where(lane_8 == 0, 1.0, 0.0).astype(jnp.float32)
    zcol5 = jnp.zeros((t_sz, 4), jnp.float32)

    xrels = []
    ohs = []
    pm1 = []
    s1 = jnp.zeros((t_sz, 8), jnp.float32)
    for k in range(_NK):
        oh = jnp.where(lane_n == idx_tile[:, k:k + 1], 1.0, 0.0)
        ohs.append(oh.astype(jnp.bfloat16))
        g0 = jnp.sum(oh * xrow0, axis=1, keepdims=True)
        g1 = jnp.sum(oh * xrow1, axis=1, keepdims=True)
        g2 = jnp.sum(oh * xrow2, axis=1, keepdims=True)
        gx = jnp.concatenate([g0, g1, g2, jnp.zeros((t_sz, 5), jnp.float32)],
                             axis=1)
        xr = gx - x_rep
        xrels.append(xr)
        pm = jax.lax.dot_general(xr.astype(jnp.bfloat16), kp_16,
                                 (((1,), (0,)), ((), ())),
                                 preferred_element_type=jnp.float32)
        if k == 0:
            pm = pm + e0
        pm = jnp.maximum(pm, 0.0)
        pm1.append(pm)
        s1 = s1 + pm
    den1 = s1 + _EPS
    pm2 = []
    s2 = jnp.zeros((t_sz, 8), jnp.float32)
    for k in range(_NK):
        p = pm1[k] / den1
        p = p * p
        pm2.append(p)
        s2 = s2 + p
    den2 = s2 + _EPS

    ya = [jnp.zeros((t_sz, c), jnp.float32) for _ in range(8)]
    yb = [jnp.zeros((t_sz, c), jnp.float32) for _ in range(8)]
    bmlp = bmlp_ref[0:1, :]
    dnum = (((1,), (0,)), ((), ()))
    for k in range(_NK):
        p = pm2[k] / den2
        p = jnp.where(p > 0.1, p, 0.0)
        xr = xrels[k]
        dis = jnp.sqrt(jnp.maximum(
            jnp.sum(xr * xr, axis=1, keepdims=True), 1e-12))
        feat7 = jnp.concatenate([x_rep[:, 0:3], xr[:, 0:3], dis, zcol5[:, 0:1]],
                                axis=1)
        xf = jax.lax.dot_general(feat7.astype(jnp.bfloat16), w7_16, dnum,
                                 preferred_element_type=jnp.float32) + bmlp
        gf = jnp.dot(ohs[k], fb16, preferred_element_type=jnp.float32)
        xf = xf.astype(jnp.bfloat16).astype(jnp.float32)
        p = p.astype(jnp.bfloat16).astype(jnp.float32)
        for j in range(8):
            pj = p[:, j:j + 1]
            ya[j] = ya[j] + gf * pj
            yb[j] = yb[j] + xf * pj

    acc = jnp.zeros((t_sz, out_c), jnp.float32)
    for j in range(8):
        acc = acc + jax.lax.dot_general(
            ya[j].astype(jnp.bfloat16),
            wr_ref[j * 2 * c:j * 2 * c + c, :].astype(jnp.bfloat16),
            dnum, preferred_element_type=jnp.float32)
        acc = acc + jax.lax.dot_general(
            yb[j].astype(jnp.bfloat16),
            wr_ref[j * 2 * c + c:(j + 1) * 2 * c, :].astype(jnp.bfloat16),
            dnum, preferred_element_type=jnp.float32)
    acc = acc + bconv_ref[0:1, :]
    acc = acc + jax.lax.dot_general(ft_tile.astype(jnp.bfloat16),
                                    wo_ref[...].astype(jnp.bfloat16),
                                    dnum, preferred_element_type=jnp.float32)
    acc = acc + bout_ref[0:1, :]
    o_ref[0] = acc


def _bn_body(x_ref, g_ref, b_ref, o_ref, *, cnt):
    xv = x_ref[...]
    s = jnp.sum(jnp.sum(xv, axis=1, keepdims=True), axis=0, keepdims=True)
    m = s / cnt
    d = xv - m
    v = jnp.sum(jnp.sum(d * d, axis=1, keepdims=True), axis=0,
                keepdims=True) / cnt
    g = jnp.reshape(g_ref[0:1, :], (1, 1, -1))
    bb = jnp.reshape(b_ref[0:1, :], (1, 1, -1))
    o_ref[...] = d / jnp.sqrt(v + 1e-5) * g + bb


def kernel(x, feature, W_mlp, b_mlp, W_conv, b_conv, W_out, b_out,
           gamma, beta, kernals):
    bsz, _, n = x.shape
    c = feature.shape[1]
    out_c = W_out.shape[0]
    ks = kernals.shape[1]
    t_sz = 256
    nt = n // t_sz

    x_p8 = jnp.pad(x, ((0, 0), (0, 5), (0, 0)))
    x_t = jnp.transpose(x_p8, (0, 2, 1))
    f_t = jnp.transpose(feature, (0, 2, 1))
    w7 = jnp.pad(W_mlp, ((0, 0), (0, 1))).T
    kp = jnp.pad(kernals, ((0, 5), (0, 0)))
    wr = jnp.transpose(W_conv.reshape(out_c, 2 * c, ks),
                       (2, 1, 0)).reshape(ks * 2 * c, out_c)
    wo = W_out.T
    row8 = lambda v: jnp.broadcast_to(v[None, :], (8, v.shape[0]))

    idx = pl.pallas_call(
        functools.partial(_knn_body, n=n),
        grid=(bsz,),
        in_specs=[pl.BlockSpec((1, 8, n), lambda b: (b, 0, 0)),
                  pl.BlockSpec((1, n, 8), lambda b: (b, 0, 0))],
        out_specs=pl.BlockSpec((1, n, 128), lambda b: (b, 0, 0)),
        out_shape=jax.ShapeDtypeStruct((bsz, n, 128), jnp.int32),
    )(x_p8, x_t)

    out_pre = pl.pallas_call(
        functools.partial(_conv_body, n=n, c=c, out_c=out_c, t_sz=t_sz),
        grid=(bsz, nt),
        in_specs=[
            pl.BlockSpec((1, t_sz, 128), lambda b, t: (b, t, 0)),
            pl.BlockSpec((1, n, c), lambda b, t: (b, 0, 0)),
            pl.BlockSpec((1, n, 8), lambda b, t: (b, 0, 0)),
            pl.BlockSpec((1, 8, n), lambda b, t: (b, 0, 0)),
            pl.BlockSpec((8, c), lambda b, t: (0, 0)),
            pl.BlockSpec((8, ks), lambda b, t: (0, 0)),
            pl.BlockSpec((ks * 2 * c, out_c), lambda b, t: (0, 0)),
            pl.BlockSpec((c, out_c), lambda b, t: (0, 0)),
            pl.BlockSpec((8, c), lambda b, t: (0, 0)),
            pl.BlockSpec((8, out_c), lambda b, t: (0, 0)),
            pl.BlockSpec((8, out_c), lambda b, t: (0, 0)),
        ],
        out_specs=pl.BlockSpec((1, t_sz, out_c), lambda b, t: (b, t, 0)),
        out_shape=jax.ShapeDtypeStruct((bsz, n, out_c), jnp.float32),
    )(idx, f_t, x_t, x_p8, w7, kp, wr, wo,
      row8(b_mlp), row8(b_conv), row8(b_out))

    out_n = pl.pallas_call(
        functools.partial(_bn_body, cnt=float(bsz * n)),
        in_specs=[
            pl.BlockSpec((bsz, n, out_c), lambda: (0, 0, 0)),
            pl.BlockSpec((8, out_c), lambda: (0, 0)),
            pl.BlockSpec((8, out_c), lambda: (0, 0)),
        ],
        out_specs=pl.BlockSpec((bsz, n, out_c), lambda: (0, 0, 0)),
        out_shape=jax.ShapeDtypeStruct((bsz, n, out_c), jnp.float32),
    )(out_pre, row8(gamma), row8(beta))

    return jnp.transpose(out_n, (0, 2, 1))


# speedup vs baseline: 1.1659x; 1.0951x over previous
import functools

import jax
import jax.numpy as jnp
from jax.experimental import pallas as pl

_NK = 20
_EPS = 1e-06


def _knn_body(x_ref, xt_ref, idx_ref, *, n):
    xr0 = x_ref[0, 0:1, :]
    xr1 = x_ref[0, 1:2, :]
    xr2 = x_ref[0, 2:3, :]
    xc0 = xt_ref[0, :, 0:1]
    xc1 = xt_ref[0, :, 1:2]
    xc2 = xt_ref[0, :, 2:3]
    sq_row = xr0 * xr0 + xr1 * xr1 + xr2 * xr2
    sq_col = xc0 * xc0 + xc1 * xc1 + xc2 * xc2
    xb16 = x_ref[0].astype(jnp.bfloat16)
    cross = jax.lax.dot_general(xb16, xb16, (((0,), (0,)), ((), ())),
                                preferred_element_type=jnp.float32)
    dist = (sq_col + sq_row) - 2.0 * cross
    row_i = jax.lax.broadcasted_iota(jnp.int32, (n, n), 0)
    col_i = jax.lax.broadcasted_iota(jnp.int32, (n, n), 1)
    neg = -dist + jnp.where(row_i == col_i, 1e3, 0.0).astype(jnp.float32)
    cols = []
    for _ in range(_NK):
        amax = jnp.argmax(neg, axis=1, keepdims=True).astype(jnp.int32)
        cols.append(amax)
        neg = jnp.where(col_i == amax, -1e30, neg)
    cols.append(jnp.zeros((n, 128 - _NK), jnp.int32))
    idx_ref[0] = jnp.concatenate(cols, axis=1)


def _conv_body(idx_ref, f_ref, x_ref, x8_ref, w7_ref, kp_ref, wr_ref, wo_ref,
               bmlp_ref, bconv_ref, bout_ref, o_ref, *, n, c, out_c, t_sz):
    t = pl.program_id(1)
    idx_tile = idx_ref[0]
    fb16 = f_ref[0].astype(jnp.bfloat16)
    x_rep = x_ref[0, pl.ds(t * t_sz, t_sz), :]
    ft_tile = f_ref[0, pl.ds(t * t_sz, t_sz), :]
    w7_16 = w7_ref[...].astype(jnp.bfloat16)
    kp_16 = kp_ref[...].astype(jnp.bfloat16)
    xrow0 = x8_ref[0, 0:1, :]
    xrow1 = x8_ref[0, 1:2, :]
    xrow2 = x8_ref[0, 2:3, :]
    lane_n = jax.lax.broadcasted_iota(jnp.int32, (t_sz, n), 1)
    lane_8 = jax.lax.broadcasted_iota(jnp.int32, (t_sz, 8), 1)
    e0 = jnp.where(lane_8 == 0, 1.0, 0.0).astype(jnp.float32)
    zcol5 = jnp.zeros((t_sz, 4), jnp.float32)

    xrels = []
    ohs = []
    pm1 = []
    s1 = jnp.zeros((t_sz, 8), jnp.float32)
    for k in range(_NK):
        oh = jnp.where(lane_n == idx_tile[:, k:k + 1], 1.0, 0.0)
        ohs.append(oh.astype(jnp.bfloat16))
        g0 = jnp.sum(oh * xrow0, axis=1, keepdims=True)
        g1 = jnp.sum(oh * xrow1, axis=1, keepdims=True)
        g2 = jnp.sum(oh * xrow2, axis=1, keepdims=True)
        gx = jnp.concatenate([g0, g1, g2, jnp.zeros((t_sz, 5), jnp.float32)],
                             axis=1)
        xr = gx - x_rep
        xrels.append(xr)
        pm = jax.lax.dot_general(xr.astype(jnp.bfloat16), kp_16,
                                 (((1,), (0,)), ((), ())),
                                 preferred_element_type=jnp.float32)
        if k == 0:
            pm = pm + e0
        pm = jnp.maximum(pm, 0.0)
        pm1.append(pm)
        s1 = s1 + pm
    den1 = s1 + _EPS
    pm2 = []
    s2 = jnp.zeros((t_sz, 8), jnp.float32)
    for k in range(_NK):
        p = pm1[k] / den1
        p = p * p
        pm2.append(p)
        s2 = s2 + p
    den2 = s2 + _EPS

    ya = [jnp.zeros((t_sz, c), jnp.float32) for _ in range(8)]
    yb = [jnp.zeros((t_sz, c), jnp.float32) for _ in range(8)]
    bmlp = bmlp_ref[0:1, :]
    dnum = (((1,), (0,)), ((), ()))
    for k in range(_NK):
        p = pm2[k] / den2
        p = jnp.where(p > 0.1, p, 0.0)
        xr = xrels[k]
        dis = jnp.sqrt(jnp.maximum(
            jnp.sum(xr * xr, axis=1, keepdims=True), 1e-12))
        feat7 = jnp.concatenate([x_rep[:, 0:3], xr[:, 0:3], dis, zcol5[:, 0:1]],
                                axis=1)
        xf = jax.lax.dot_general(feat7.astype(jnp.bfloat16), w7_16, dnum,
                                 preferred_element_type=jnp.float32) + bmlp
        gf = jnp.dot(ohs[k], fb16, preferred_element_type=jnp.float32)
        xf = xf.astype(jnp.bfloat16).astype(jnp.float32)
        p = p.astype(jnp.bfloat16).astype(jnp.float32)
        for j in range(8):
            pj = p[:, j:j + 1]
            ya[j] = ya[j] + gf * pj
            yb[j] = yb[j] + xf * pj

    acc = jnp.zeros((t_sz, out_c), jnp.float32)
    for j in range(8):
        acc = acc + jax.lax.dot_general(
            ya[j].astype(jnp.bfloat16),
            wr_ref[j * 2 * c:j * 2 * c + c, :].astype(jnp.bfloat16),
            dnum, preferred_element_type=jnp.float32)
        acc = acc + jax.lax.dot_general(
            yb[j].astype(jnp.bfloat16),
            wr_ref[j * 2 * c + c:(j + 1) * 2 * c, :].astype(jnp.bfloat16),
            dnum, preferred_element_type=jnp.float32)
    acc = acc + bconv_ref[0:1, :]
    acc = acc + jax.lax.dot_general(ft_tile.astype(jnp.bfloat16),
                                    wo_ref[...].astype(jnp.bfloat16),
                                    dnum, preferred_element_type=jnp.float32)
    acc = acc + bout_ref[0:1, :]
    o_ref[0] = acc


def _bn_body(x_ref, g_ref, b_ref, o_ref, *, cnt):
    xv = x_ref[...]
    s = jnp.sum(jnp.sum(xv, axis=1, keepdims=True), axis=0, keepdims=True)
    m = s / cnt
    d = xv - m
    v = jnp.sum(jnp.sum(d * d, axis=1, keepdims=True), axis=0,
                keepdims=True) / cnt
    g = jnp.reshape(g_ref[0:1, :], (1, 1, -1))
    bb = jnp.reshape(b_ref[0:1, :], (1, 1, -1))
    o_ref[...] = d / jnp.sqrt(v + 1e-5) * g + bb


def kernel(x, feature, W_mlp, b_mlp, W_conv, b_conv, W_out, b_out,
           gamma, beta, kernals):
    bsz, _, n = x.shape
    c = feature.shape[1]
    out_c = W_out.shape[0]
    ks = kernals.shape[1]
    t_sz = 512
    nt = n // t_sz

    x_p8 = jnp.pad(x, ((0, 0), (0, 5), (0, 0)))
    x_t = jnp.transpose(x_p8, (0, 2, 1))
    f_t = jnp.transpose(feature, (0, 2, 1))
    w7 = jnp.pad(W_mlp, ((0, 0), (0, 1))).T
    kp = jnp.pad(kernals, ((0, 5), (0, 0)))
    wr = jnp.transpose(W_conv.reshape(out_c, 2 * c, ks),
                       (2, 1, 0)).reshape(ks * 2 * c, out_c)
    wo = W_out.T
    row8 = lambda v: jnp.broadcast_to(v[None, :], (8, v.shape[0]))

    idx = pl.pallas_call(
        functools.partial(_knn_body, n=n),
        grid=(bsz,),
        in_specs=[pl.BlockSpec((1, 8, n), lambda b: (b, 0, 0)),
                  pl.BlockSpec((1, n, 8), lambda b: (b, 0, 0))],
        out_specs=pl.BlockSpec((1, n, 128), lambda b: (b, 0, 0)),
        out_shape=jax.ShapeDtypeStruct((bsz, n, 128), jnp.int32),
    )(x_p8, x_t)

    out_pre = pl.pallas_call(
        functools.partial(_conv_body, n=n, c=c, out_c=out_c, t_sz=t_sz),
        grid=(bsz, nt),
        in_specs=[
            pl.BlockSpec((1, t_sz, 128), lambda b, t: (b, t, 0)),
            pl.BlockSpec((1, n, c), lambda b, t: (b, 0, 0)),
            pl.BlockSpec((1, n, 8), lambda b, t: (b, 0, 0)),
            pl.BlockSpec((1, 8, n), lambda b, t: (b, 0, 0)),
            pl.BlockSpec((8, c), lambda b, t: (0, 0)),
            pl.BlockSpec((8, ks), lambda b, t: (0, 0)),
            pl.BlockSpec((ks * 2 * c, out_c), lambda b, t: (0, 0)),
            pl.BlockSpec((c, out_c), lambda b, t: (0, 0)),
            pl.BlockSpec((8, c), lambda b, t: (0, 0)),
            pl.BlockSpec((8, out_c), lambda b, t: (0, 0)),
            pl.BlockSpec((8, out_c), lambda b, t: (0, 0)),
        ],
        out_specs=pl.BlockSpec((1, t_sz, out_c), lambda b, t: (b, t, 0)),
        out_shape=jax.ShapeDtypeStruct((bsz, n, out_c), jnp.float32),
    )(idx, f_t, x_t, x_p8, w7, kp, wr, wo,
      row8(b_mlp), row8(b_conv), row8(b_out))

    out_n = pl.pallas_call(
        functools.partial(_bn_body, cnt=float(bsz * n)),
        in_specs=[
            pl.BlockSpec((bsz, n, out_c), lambda: (0, 0, 0)),
            pl.BlockSpec((8, out_c), lambda: (0, 0)),
            pl.BlockSpec((8, out_c), lambda: (0, 0)),
        ],
        out_specs=pl.BlockSpec((bsz, n, out_c), lambda: (0, 0, 0)),
        out_shape=jax.ShapeDtypeStruct((bsz, n, out_c), jnp.float32),
    )(out_pre, row8(gamma), row8(beta))

    return jnp.transpose(out_n, (0, 2, 1))
